# baseline (speedup 1.0000x reference)
import functools

import jax
import jax.numpy as jnp
from jax import lax
from jax.experimental import pallas as pl
from jax.experimental.pallas import tpu as pltpu
from jax.experimental.pallas import tpu_sc as plsc

N_FIELDS = 26
VOCAB = 100000
D_TOKEN = 32
BATCH = 4096

NUM_CORES = 2
NUM_SUBCORES = 16
NW = NUM_CORES * NUM_SUBCORES
B_PER_W = BATCH // NW


def _sc_gather(tab, x_t):
    mesh = plsc.VectorSubcoreMesh(core_axis_name="c", subcore_axis_name="s")

    @functools.partial(
        pl.kernel,
        mesh=mesh,
        out_type=jax.ShapeDtypeStruct((N_FIELDS, BATCH, D_TOKEN), jnp.float32),
        compiler_params=pltpu.CompilerParams(use_tc_tiling_on_sc=False),
        scratch_types=[
            pltpu.VMEM((N_FIELDS, B_PER_W), jnp.int32),
            pltpu.VMEM((N_FIELDS, B_PER_W, D_TOKEN), jnp.float32),
            pltpu.SemaphoreType.DMA,
            pltpu.SemaphoreType.DMA,
        ],
    )
    def k(tab_hbm, xt_hbm, out_hbm, idx_v, rows_v, sem0, sem1):
        wid = lax.axis_index("s") * NUM_CORES + lax.axis_index("c")
        base = wid * B_PER_W

        pltpu.sync_copy(xt_hbm.at[:, pl.ds(base, B_PER_W)], idx_v)

        def fire(f, carry):
            pltpu.async_copy(
                tab_hbm.at[f].at[idx_v.at[f]],
                rows_v.at[f],
                sem0,
            )
            return carry

        lax.fori_loop(0, N_FIELDS, fire, 0)
        pltpu.make_async_copy(
            tab_hbm.at[pl.ds(0, N_FIELDS), pl.ds(0, B_PER_W)],
            rows_v,
            sem0,
        ).wait()

        def wout(f, carry):
            pltpu.async_copy(
                rows_v.at[pl.ds(f, 1)],
                out_hbm.at[pl.ds(f, 1), pl.ds(base, B_PER_W)],
                sem1,
            )
            return carry

        lax.fori_loop(0, N_FIELDS, wout, 0)

        def wdrain(f, carry):
            pltpu.make_async_copy(
                rows_v.at[pl.ds(f, 1)],
                out_hbm.at[pl.ds(f, 1), pl.ds(base, B_PER_W)],
                sem1,
            ).wait()
            return carry

        lax.fori_loop(0, N_FIELDS, wdrain, 0)

    return k(tab, x_t)


def kernel(x_cat, tables):
    x_t = x_cat.T
    out = _sc_gather(tables, x_t)
    return jnp.transpose(out, (1, 0, 2))
